# Initial kernel scaffold; baseline (speedup 1.0000x reference)
import jax
import jax.numpy as jnp
from jax.experimental import pallas as pl


def kernel(node_feats, edge_feats, src, dst, w_src_gate, b_src_gate, w_dst_gate, b_dst_gate, w_edge_gate, b_edge_gate, w_src_update, b_src_update, w_dst_update, b_dst_update, bn_nodes_gamma, bn_nodes_beta, bn_edges_gamma, bn_edges_beta):
    raise NotImplementedError("write your pallas kernel here")


# speedup vs baseline: 1.0585x; 1.0585x over previous
import functools

import jax
import jax.numpy as jnp
from jax import lax
from jax.experimental import pallas as pl
from jax.experimental.pallas import tpu as pltpu

_DIV_EPS = 1e-6
_BN_EPS = 1e-5


def _recip(x):
    r = pl.reciprocal(x, approx=True)
    return r * (2.0 - x * r)


def _pass1_kernel(node_ref, edge_ref, src_ref, dst_ref, w3_ref, b3_ref,
                  weg_ref, beg_ref, m_out, acc_out, st_out, nl_ref,
                  *, num_edges, tiles_per_core, edge_tile):
    f32 = jnp.float32
    bf16 = jnp.bfloat16
    N = node_ref.shape[0]
    F = weg_ref.shape[0]
    TE = edge_tile
    c = pl.program_id(0)
    t = pl.program_id(1)

    @pl.when(t == 0)
    def _init():
        nl = jnp.dot(node_ref[...], w3_ref[...],
                     preferred_element_type=f32) + b3_ref[...]
        nl_ref[...] = nl.astype(bf16)
        acc_out[...] = jnp.zeros_like(acc_out)
        st_out[...] = jnp.zeros_like(st_out)

    lane = lax.broadcasted_iota(jnp.int32, (TE, N), 1)
    S = (lane == src_ref[...]).astype(bf16)
    D = (lane == dst_ref[...]).astype(bf16)

    base = (c * tiles_per_core + t) * TE
    eidx = base + lax.broadcasted_iota(jnp.int32, (TE, 1), 0)
    valid = (eidx < num_edges).astype(f32)

    gat = jnp.dot(S, nl_ref[:, 0:2 * F], preferred_element_type=f32)
    edst = jnp.dot(D, nl_ref[:, 2 * F:3 * F], preferred_element_type=f32)
    eg = jnp.dot(edge_ref[...].astype(bf16), weg_ref[...],
                 preferred_element_type=f32)
    m = gat[:, 0:F] + edst + eg + beg_ref[...]
    m_out[...] = m.astype(bf16)

    mv = m * valid
    st_out[0, 0:1, :] += jnp.sum(mv, axis=0, keepdims=True)
    st_out[0, 1:2, :] += jnp.sum(mv * m, axis=0, keepdims=True)

    e = jnp.exp(-jnp.abs(m))
    r = _recip(1.0 + e)
    sigma = jnp.where(m >= 0, r, e * r) * valid

    contrib = jnp.concatenate([gat[:, F:2 * F] * sigma, sigma],
                              axis=1).astype(bf16)
    dn = (((0,), (0,)), ((), ()))
    acc_out[0] += lax.dot_general(D, contrib, dn, preferred_element_type=f32)


def _finalize_kernel(node_ref, nodebf_ref, acc_ref, st_ref, wsu_ref, bsu_ref,
                     gn_ref, bn_ref, ge_ref, be_ref, x_out, ss_out,
                     *, num_edges):
    f32 = jnp.float32
    N, F = node_ref.shape
    xl = jnp.dot(nodebf_ref[...], wsu_ref[...],
                 preferred_element_type=f32) + bsu_ref[...]
    num = acc_ref[0, :, 0:F] + acc_ref[1, :, 0:F]
    den = acc_ref[0, :, F:2 * F] + acc_ref[1, :, F:2 * F]
    x = xl + num * _recip(den + _DIV_EPS)
    mean = jnp.mean(x, axis=0, keepdims=True)
    var = jnp.mean((x - mean) ** 2, axis=0, keepdims=True)
    xn = (x - mean) * lax.rsqrt(var + _BN_EPS) * gn_ref[...] + bn_ref[...]
    x_out[...] = node_ref[...] + jnp.clip(xn, 0.0, 6.0)

    inv_e = 1.0 / float(num_edges)
    s = (st_ref[0, 0:1, :] + st_ref[1, 0:1, :]) * inv_e
    q = (st_ref[0, 1:2, :] + st_ref[1, 1:2, :]) * inv_e
    var_e = q - s * s
    scale = ge_ref[...] * lax.rsqrt(var_e + _BN_EPS)
    ss_out[0:1, :] = scale
    ss_out[1:2, :] = be_ref[...] - s * scale
    ss_out[2:8, :] = jnp.zeros((6, F), f32)


def _edge_out_kernel(edge_ref, m_ref, ss_ref, y_out):
    y_out[...] = edge_ref[...] + jnp.clip(
        m_ref[...].astype(jnp.float32) * ss_ref[0:1, :] + ss_ref[1:2, :],
        0.0, 6.0)


def _row_tile(n, cap=1024, step=8):
    best = min(n, step)
    d = step
    while d <= min(n, cap):
        if n % d == 0:
            best = d
        d += step
    return best


def kernel(node_feats, edge_feats, src, dst,
           w_src_gate, b_src_gate, w_dst_gate, b_dst_gate,
           w_edge_gate, b_edge_gate, w_src_update, b_src_update,
           w_dst_update, b_dst_update,
           bn_nodes_gamma, bn_nodes_beta, bn_edges_gamma, bn_edges_beta):
    f32 = jnp.float32
    bf16 = jnp.bfloat16
    N, F = node_feats.shape
    E = edge_feats.shape[0]
    TE = 512
    CH = 2 * TE
    E_pad = ((E + CH - 1) // CH) * CH
    pad = E_pad - E
    edge_p = jnp.pad(edge_feats, ((0, pad), (0, 0)))
    src_p = jnp.pad(src.astype(jnp.int32), (0, pad)).reshape(E_pad, 1)
    dst_p = jnp.pad(dst.astype(jnp.int32), (0, pad)).reshape(E_pad, 1)
    Th = E_pad // TE // 2

    node_bf = node_feats.astype(bf16)
    w3 = jnp.concatenate([w_src_gate, w_dst_update, w_dst_gate],
                         axis=1).astype(bf16)
    b3 = jnp.concatenate([b_src_gate, b_dst_update, b_dst_gate], axis=1)

    m, acc, st = pl.pallas_call(
        functools.partial(_pass1_kernel, num_edges=E, tiles_per_core=Th,
                          edge_tile=TE),
        out_shape=(jax.ShapeDtypeStruct((E_pad, F), bf16),
                   jax.ShapeDtypeStruct((2, N, 2 * F), f32),
                   jax.ShapeDtypeStruct((2, 8, F), f32)),
        grid=(2, Th),
        in_specs=[
            pl.BlockSpec((N, F), lambda c, t: (0, 0)),
            pl.BlockSpec((TE, F), lambda c, t: (c * Th + t, 0)),
            pl.BlockSpec((TE, 1), lambda c, t: (c * Th + t, 0)),
            pl.BlockSpec((TE, 1), lambda c, t: (c * Th + t, 0)),
            pl.BlockSpec((F, 3 * F), lambda c, t: (0, 0)),
            pl.BlockSpec((1, 3 * F), lambda c, t: (0, 0)),
            pl.BlockSpec((F, F), lambda c, t: (0, 0)),
            pl.BlockSpec((1, F), lambda c, t: (0, 0)),
        ],
        out_specs=(
            pl.BlockSpec((TE, F), lambda c, t: (c * Th + t, 0)),
            pl.BlockSpec((1, N, 2 * F), lambda c, t: (c, 0, 0)),
            pl.BlockSpec((1, 8, F), lambda c, t: (c, 0, 0)),
        ),
        scratch_shapes=[pltpu.VMEM((N, 3 * F), bf16)],
        compiler_params=pltpu.CompilerParams(
            dimension_semantics=("parallel", "arbitrary"),
            vmem_limit_bytes=48 * 1024 * 1024),
    )(node_bf, edge_p, src_p, dst_p, w3, b3,
      w_edge_gate.astype(bf16), b_edge_gate)

    x, ss = pl.pallas_call(
        functools.partial(_finalize_kernel, num_edges=E),
        out_shape=(jax.ShapeDtypeStruct((N, F), f32),
                   jax.ShapeDtypeStruct((8, F), f32)),
    )(node_feats, node_bf, acc, st, w_src_update.astype(bf16), b_src_update,
      bn_nodes_gamma, bn_nodes_beta, bn_edges_gamma, bn_edges_beta)

    E4 = E_pad // 4
    edge4 = edge_p.reshape(E4, 4 * F)
    m4 = m.reshape(E4, 4 * F)
    ss4 = jnp.tile(ss, (1, 4))
    tb = _row_tile(E4, cap=1024)
    y4 = pl.pallas_call(
        _edge_out_kernel,
        out_shape=jax.ShapeDtypeStruct((E4, 4 * F), f32),
        grid=(E4 // tb,),
        in_specs=[
            pl.BlockSpec((tb, 4 * F), lambda t: (t, 0)),
            pl.BlockSpec((tb, 4 * F), lambda t: (t, 0)),
            pl.BlockSpec((8, 4 * F), lambda t: (0, 0)),
        ],
        out_specs=pl.BlockSpec((tb, 4 * F), lambda t: (t, 0)),
        compiler_params=pltpu.CompilerParams(
            dimension_semantics=("parallel",)),
    )(edge4, m4, ss4)
    y = y4.reshape(E_pad, F)[:E]
    return x, y


# speedup vs baseline: 1.1465x; 1.0832x over previous
import functools

import jax
import jax.numpy as jnp
from jax import lax
from jax.experimental import pallas as pl
from jax.experimental.pallas import tpu as pltpu

_DIV_EPS = 1e-6
_BN_EPS = 1e-5


def _recip(x):
    r = pl.reciprocal(x, approx=True)
    return r * (2.0 - x * r)


def _pass1_kernel(nodebf_ref, node_ref, edge_ref, src_ref, dst_ref,
                  w4_ref, b4_ref, weg_ref, beg_ref,
                  gn_ref, bn_ref, ge_ref, be_ref,
                  m_out, x_out, ss_out,
                  nl_ref, xl_ref, acc_ref, st_ref,
                  *, num_edges, edge_tile, padded):
    f32 = jnp.float32
    bf16 = jnp.bfloat16
    N = nodebf_ref.shape[0]
    F = weg_ref.shape[0]
    TE = edge_tile
    t = pl.program_id(0)
    last = pl.num_programs(0) - 1

    @pl.when(t == 0)
    def _init():
        nl4 = jnp.dot(nodebf_ref[...], w4_ref[...],
                      preferred_element_type=f32) + b4_ref[...]
        nl_ref[...] = nl4[:, 0:3 * F].astype(bf16)
        xl_ref[...] = nl4[:, 3 * F:4 * F]
        acc_ref[...] = jnp.zeros_like(acc_ref)
        st_ref[...] = jnp.zeros_like(st_ref)

    lane = lax.broadcasted_iota(jnp.int32, (TE, N), 1)
    S = (lane == src_ref[...]).astype(bf16)
    D = (lane == dst_ref[...]).astype(bf16)

    gat = jnp.dot(S, nl_ref[:, 0:2 * F], preferred_element_type=f32)
    edst = jnp.dot(D, nl_ref[:, 2 * F:3 * F], preferred_element_type=f32)
    eg = jnp.dot(edge_ref[...].astype(bf16), weg_ref[...],
                 preferred_element_type=f32)
    m = gat[:, 0:F] + edst + eg + beg_ref[...]
    m_out[...] = m.astype(bf16)

    if padded:
        eidx = t * TE + lax.broadcasted_iota(jnp.int32, (TE, 1), 0)
        valid = (eidx < num_edges).astype(f32)
        mv = m * valid
    else:
        valid = None
        mv = m

    st_ref[0:1, :] += jnp.sum(mv, axis=0, keepdims=True)
    st_ref[1:2, :] += jnp.sum(mv * m, axis=0, keepdims=True)

    e = jnp.exp(-jnp.abs(m))
    r = _recip(1.0 + e)
    sigma = jnp.where(m >= 0, r, e * r)
    if padded:
        sigma = sigma * valid

    contrib = jnp.concatenate([gat[:, F:2 * F] * sigma, sigma],
                              axis=1).astype(bf16)
    dn = (((0,), (0,)), ((), ()))
    acc_ref[...] += lax.dot_general(D, contrib, dn, preferred_element_type=f32)

    @pl.when(t == last)
    def _finalize():
        num = acc_ref[:, 0:F]
        den = acc_ref[:, F:2 * F]
        x = xl_ref[...] + num * _recip(den + _DIV_EPS)
        mean = jnp.mean(x, axis=0, keepdims=True)
        var = jnp.mean((x - mean) ** 2, axis=0, keepdims=True)
        xn = (x - mean) * lax.rsqrt(var + _BN_EPS) * gn_ref[...] + bn_ref[...]
        x_out[...] = node_ref[...] + jnp.clip(xn, 0.0, 6.0)

        inv_e = 1.0 / float(num_edges)
        s = st_ref[0:1, :] * inv_e
        q = st_ref[1:2, :] * inv_e
        var_e = q - s * s
        scale = ge_ref[...] * lax.rsqrt(var_e + _BN_EPS)
        ss_out[0:1, :] = scale
        ss_out[1:2, :] = be_ref[...] - s * scale
        ss_out[2:8, :] = jnp.zeros((6, F), f32)


def _edge_out_kernel(edge_ref, m_ref, ss_ref, y_out):
    y_out[...] = edge_ref[...] + jnp.clip(
        m_ref[...].astype(jnp.float32) * ss_ref[0:1, :] + ss_ref[1:2, :],
        0.0, 6.0)


def _pick_tile(n, cap, step=8):
    best = 0
    d = step
    while d <= min(n, cap):
        if n % d == 0:
            best = d
        d += step
    return best


def kernel(node_feats, edge_feats, src, dst,
           w_src_gate, b_src_gate, w_dst_gate, b_dst_gate,
           w_edge_gate, b_edge_gate, w_src_update, b_src_update,
           w_dst_update, b_dst_update,
           bn_nodes_gamma, bn_nodes_beta, bn_edges_gamma, bn_edges_beta):
    f32 = jnp.float32
    bf16 = jnp.bfloat16
    N, F = node_feats.shape
    E = edge_feats.shape[0]

    TE = _pick_tile(E, cap=2048)
    if TE >= 256:
        E_pad, padded = E, False
        edge_p, src_p, dst_p = edge_feats, src, dst
    else:
        TE = 1024
        E_pad = ((E + TE - 1) // TE) * TE
        padded = True
        edge_p = jnp.pad(edge_feats, ((0, E_pad - E), (0, 0)))
        src_p = jnp.pad(src, (0, E_pad - E))
        dst_p = jnp.pad(dst, (0, E_pad - E))
    src_p = src_p.astype(jnp.int32).reshape(E_pad, 1)
    dst_p = dst_p.astype(jnp.int32).reshape(E_pad, 1)
    n_tiles = E_pad // TE

    node_bf = node_feats.astype(bf16)
    w4 = jnp.concatenate([w_src_gate, w_dst_update, w_dst_gate, w_src_update],
                         axis=1).astype(bf16)
    b4 = jnp.concatenate([b_src_gate, b_dst_update, b_dst_gate, b_src_update],
                         axis=1)

    m, x, ss = pl.pallas_call(
        functools.partial(_pass1_kernel, num_edges=E, edge_tile=TE,
                          padded=padded),
        out_shape=(jax.ShapeDtypeStruct((E_pad, F), bf16),
                   jax.ShapeDtypeStruct((N, F), f32),
                   jax.ShapeDtypeStruct((8, F), f32)),
        grid=(n_tiles,),
        in_specs=[
            pl.BlockSpec((N, F), lambda t: (0, 0)),
            pl.BlockSpec((N, F), lambda t: (0, 0)),
            pl.BlockSpec((TE, F), lambda t: (t, 0)),
            pl.BlockSpec((TE, 1), lambda t: (t, 0)),
            pl.BlockSpec((TE, 1), lambda t: (t, 0)),
            pl.BlockSpec((F, 4 * F), lambda t: (0, 0)),
            pl.BlockSpec((1, 4 * F), lambda t: (0, 0)),
            pl.BlockSpec((F, F), lambda t: (0, 0)),
            pl.BlockSpec((1, F), lambda t: (0, 0)),
            pl.BlockSpec((1, F), lambda t: (0, 0)),
            pl.BlockSpec((1, F), lambda t: (0, 0)),
            pl.BlockSpec((1, F), lambda t: (0, 0)),
            pl.BlockSpec((1, F), lambda t: (0, 0)),
        ],
        out_specs=(
            pl.BlockSpec((TE, F), lambda t: (t, 0)),
            pl.BlockSpec((N, F), lambda t: (0, 0)),
            pl.BlockSpec((8, F), lambda t: (0, 0)),
        ),
        scratch_shapes=[pltpu.VMEM((N, 3 * F), bf16),
                        pltpu.VMEM((N, F), f32),
                        pltpu.VMEM((N, 2 * F), f32),
                        pltpu.VMEM((8, F), f32)],
        compiler_params=pltpu.CompilerParams(
            dimension_semantics=("arbitrary",),
            vmem_limit_bytes=56 * 1024 * 1024),
    )(node_bf, node_feats, edge_p, src_p, dst_p, w4, b4,
      w_edge_gate.astype(bf16), b_edge_gate,
      bn_nodes_gamma, bn_nodes_beta, bn_edges_gamma, bn_edges_beta)

    E4 = E_pad // 4
    edge4 = edge_p.reshape(E4, 4 * F)
    m4 = m.reshape(E4, 4 * F)
    ss4 = jnp.tile(ss, (1, 4))
    tb = _pick_tile(E4, cap=1024)
    if tb == 0:
        tb = E4
    y4 = pl.pallas_call(
        _edge_out_kernel,
        out_shape=jax.ShapeDtypeStruct((E4, 4 * F), f32),
        grid=(E4 // tb,),
        in_specs=[
            pl.BlockSpec((tb, 4 * F), lambda t: (t, 0)),
            pl.BlockSpec((tb, 4 * F), lambda t: (t, 0)),
            pl.BlockSpec((8, 4 * F), lambda t: (0, 0)),
        ],
        out_specs=pl.BlockSpec((tb, 4 * F), lambda t: (t, 0)),
        compiler_params=pltpu.CompilerParams(
            dimension_semantics=("arbitrary",)),
    )(edge4, m4, ss4)
    y = y4.reshape(E_pad, F)[:E]
    return x, y


# speedup vs baseline: 1.5822x; 1.3800x over previous
import functools

import jax
import jax.numpy as jnp
from jax import lax
from jax.experimental import pallas as pl
from jax.experimental.pallas import tpu as pltpu

_DIV_EPS = 1e-6
_BN_EPS = 1e-5


def _recip(x):
    r = pl.reciprocal(x, approx=True)
    return r * (2.0 - x * r)


def _pass1_kernel(nodebf_ref, node_ref, edge_ref, src_ref, dstr_ref,
                  wa_ref, wdg_ref, weg_ref, bm_ref, bdu_ref,
                  wsu_ref, bsu_ref,
                  gn_ref, bn_ref, ge_ref, be_ref,
                  m_out, x_out, ss_out,
                  acc_ref, st_ref,
                  *, num_edges, edge_tile, padded):
    f32 = jnp.float32
    bf16 = jnp.bfloat16
    N = nodebf_ref.shape[0]
    F = wdg_ref.shape[0]
    TE = edge_tile
    t = pl.program_id(0)
    last = pl.num_programs(0) - 1

    @pl.when(t == 0)
    def _init():
        acc_ref[...] = jnp.zeros_like(acc_ref)
        st_ref[...] = jnp.zeros_like(st_ref)

    lane = lax.broadcasted_iota(jnp.int32, (TE, N), 1)
    S = (lane == src_ref[...]).astype(bf16)
    subl = lax.broadcasted_iota(jnp.int32, (N, TE), 0)
    Dt = (subl == dstr_ref[0]).astype(bf16)

    gS = jnp.dot(S, nodebf_ref[...], preferred_element_type=f32)
    dnt = (((0,), (0,)), ((), ()))
    gD = lax.dot_general(Dt, nodebf_ref[...], dnt,
                         preferred_element_type=f32)

    gSb = gS.astype(bf16)
    P1 = jnp.dot(gSb, wa_ref[...], preferred_element_type=f32)
    P2 = jnp.dot(gD.astype(bf16), wdg_ref[...], preferred_element_type=f32)
    P3 = jnp.dot(edge_ref[...].astype(bf16), weg_ref[...],
                 preferred_element_type=f32)
    m = P1[:, 0:F] + P2 + P3 + bm_ref[...]
    m_out[...] = m.astype(bf16)

    if padded:
        eidx = t * TE + lax.broadcasted_iota(jnp.int32, (TE, 1), 0)
        valid = (eidx < num_edges).astype(f32)
        mv = m * valid
    else:
        valid = None
        mv = m

    st_ref[0:1, :] += jnp.sum(mv, axis=0, keepdims=True)
    st_ref[1:2, :] += jnp.sum(mv * m, axis=0, keepdims=True)

    e = jnp.exp(-jnp.abs(m))
    r = _recip(1.0 + e)
    sigma = jnp.where(m >= 0, r, e * r)
    if padded:
        sigma = sigma * valid

    Bh = P1[:, F:2 * F] + bdu_ref[...]
    contrib = jnp.concatenate([Bh * sigma, sigma], axis=1).astype(bf16)
    acc_ref[...] += jnp.dot(Dt, contrib, preferred_element_type=f32)

    @pl.when(t == last)
    def _finalize():
        xl = jnp.dot(nodebf_ref[...], wsu_ref[...],
                     preferred_element_type=f32) + bsu_ref[...]
        num = acc_ref[:, 0:F]
        den = acc_ref[:, F:2 * F]
        x = xl + num * _recip(den + _DIV_EPS)
        mean = jnp.mean(x, axis=0, keepdims=True)
        var = jnp.mean((x - mean) ** 2, axis=0, keepdims=True)
        xn = (x - mean) * lax.rsqrt(var + _BN_EPS) * gn_ref[...] + bn_ref[...]
        x_out[...] = node_ref[...] + jnp.clip(xn, 0.0, 6.0)

        inv_e = 1.0 / float(num_edges)
        s = st_ref[0:1, :] * inv_e
        q = st_ref[1:2, :] * inv_e
        var_e = q - s * s
        scale = ge_ref[...] * lax.rsqrt(var_e + _BN_EPS)
        ss_out[0:1, :] = scale
        ss_out[1:2, :] = be_ref[...] - s * scale
        ss_out[2:8, :] = jnp.zeros((6, F), f32)


def _edge_out_kernel(edge_ref, m_ref, ss_ref, y_out):
    y_out[...] = edge_ref[...] + jnp.clip(
        m_ref[...].astype(jnp.float32) * ss_ref[0:1, :] + ss_ref[1:2, :],
        0.0, 6.0)


def _pick_tile(n, cap, step=8):
    best = 0
    d = step
    while d <= min(n, cap):
        if n % d == 0:
            best = d
        d += step
    return best


def kernel(node_feats, edge_feats, src, dst,
           w_src_gate, b_src_gate, w_dst_gate, b_dst_gate,
           w_edge_gate, b_edge_gate, w_src_update, b_src_update,
           w_dst_update, b_dst_update,
           bn_nodes_gamma, bn_nodes_beta, bn_edges_gamma, bn_edges_beta):
    f32 = jnp.float32
    bf16 = jnp.bfloat16
    N, F = node_feats.shape
    E = edge_feats.shape[0]

    TE = _pick_tile(E, cap=2048)
    if TE >= 256:
        E_pad, padded = E, False
        edge_p, src_p, dst_p = edge_feats, src, dst
    else:
        TE = 1024
        E_pad = ((E + TE - 1) // TE) * TE
        padded = True
        edge_p = jnp.pad(edge_feats, ((0, E_pad - E), (0, 0)))
        src_p = jnp.pad(src, (0, E_pad - E))
        dst_p = jnp.pad(dst, (0, E_pad - E))
    src_c = src_p.astype(jnp.int32).reshape(E_pad, 1)
    dst_r = dst_p.astype(jnp.int32).reshape(E_pad // TE, 1, TE)
    n_tiles = E_pad // TE

    node_bf = node_feats.astype(bf16)
    wa = jnp.concatenate([w_src_gate, w_dst_update], axis=1).astype(bf16)
    bm = b_src_gate + b_dst_gate + b_edge_gate

    m, x, ss = pl.pallas_call(
        functools.partial(_pass1_kernel, num_edges=E, edge_tile=TE,
                          padded=padded),
        out_shape=(jax.ShapeDtypeStruct((E_pad, F), bf16),
                   jax.ShapeDtypeStruct((N, F), f32),
                   jax.ShapeDtypeStruct((8, F), f32)),
        grid=(n_tiles,),
        in_specs=[
            pl.BlockSpec((N, F), lambda t: (0, 0)),
            pl.BlockSpec((N, F), lambda t: (0, 0)),
            pl.BlockSpec((TE, F), lambda t: (t, 0)),
            pl.BlockSpec((TE, 1), lambda t: (t, 0)),
            pl.BlockSpec((1, 1, TE), lambda t: (t, 0, 0)),
            pl.BlockSpec((F, 2 * F), lambda t: (0, 0)),
            pl.BlockSpec((F, F), lambda t: (0, 0)),
            pl.BlockSpec((F, F), lambda t: (0, 0)),
            pl.BlockSpec((1, F), lambda t: (0, 0)),
            pl.BlockSpec((1, F), lambda t: (0, 0)),
            pl.BlockSpec((F, F), lambda t: (0, 0)),
            pl.BlockSpec((1, F), lambda t: (0, 0)),
            pl.BlockSpec((1, F), lambda t: (0, 0)),
            pl.BlockSpec((1, F), lambda t: (0, 0)),
            pl.BlockSpec((1, F), lambda t: (0, 0)),
            pl.BlockSpec((1, F), lambda t: (0, 0)),
        ],
        out_specs=(
            pl.BlockSpec((TE, F), lambda t: (t, 0)),
            pl.BlockSpec((N, F), lambda t: (0, 0)),
            pl.BlockSpec((8, F), lambda t: (0, 0)),
        ),
        scratch_shapes=[pltpu.VMEM((N, 2 * F), f32),
                        pltpu.VMEM((8, F), f32)],
        compiler_params=pltpu.CompilerParams(
            dimension_semantics=("arbitrary",),
            vmem_limit_bytes=56 * 1024 * 1024),
    )(node_bf, node_feats, edge_p, src_c, dst_r,
      wa, w_dst_gate.astype(bf16), w_edge_gate.astype(bf16), bm, b_dst_update,
      w_src_update.astype(bf16), b_src_update,
      bn_nodes_gamma, bn_nodes_beta, bn_edges_gamma, bn_edges_beta)

    E4 = E_pad // 4
    edge4 = edge_p.reshape(E4, 4 * F)
    m4 = m.reshape(E4, 4 * F)
    ss4 = jnp.tile(ss, (1, 4))
    tb = _pick_tile(E4, cap=1024)
    if tb == 0:
        tb = E4
    y4 = pl.pallas_call(
        _edge_out_kernel,
        out_shape=jax.ShapeDtypeStruct((E4, 4 * F), f32),
        grid=(E4 // tb,),
        in_specs=[
            pl.BlockSpec((tb, 4 * F), lambda t: (t, 0)),
            pl.BlockSpec((tb, 4 * F), lambda t: (t, 0)),
            pl.BlockSpec((8, 4 * F), lambda t: (0, 0)),
        ],
        out_specs=pl.BlockSpec((tb, 4 * F), lambda t: (t, 0)),
        compiler_params=pltpu.CompilerParams(
            dimension_semantics=("arbitrary",)),
    )(edge4, m4, ss4)
    y = y4.reshape(E_pad, F)[:E]
    return x, y


# speedup vs baseline: 1.6420x; 1.0378x over previous
import functools

import jax
import jax.numpy as jnp
from jax import lax
from jax.experimental import pallas as pl
from jax.experimental.pallas import tpu as pltpu

_DIV_EPS = 1e-6
_BN_EPS = 1e-5


def _recip(x):
    r = pl.reciprocal(x, approx=True)
    return r * (2.0 - x * r)


def _pass1_kernel(nodebf_ref, node_ref, edge_ref, src_ref, dstr_ref,
                  wa_ref, wdg_ref, weg_ref, bm_ref, bdu_ref,
                  wsu_ref, bsu_ref,
                  gn_ref, bn_ref, ge_ref, be_ref,
                  m_out, x_out, ss_out,
                  acc_ref, st_ref,
                  *, num_edges, edge_tile, padded):
    f32 = jnp.float32
    bf16 = jnp.bfloat16
    N = nodebf_ref.shape[0]
    F = wdg_ref.shape[0]
    TE = edge_tile
    t = pl.program_id(0)
    last = pl.num_programs(0) - 1

    @pl.when(t == 0)
    def _init():
        acc_ref[...] = jnp.zeros_like(acc_ref)
        st_ref[...] = jnp.zeros_like(st_ref)

    subl = lax.broadcasted_iota(jnp.int32, (N, TE), 0)
    St = (subl == src_ref[0]).astype(bf16)
    Dt = (subl == dstr_ref[0]).astype(bf16)

    dnt = (((0,), (0,)), ((), ()))
    gS = lax.dot_general(St, nodebf_ref[...], dnt,
                         preferred_element_type=f32)
    gD = lax.dot_general(Dt, nodebf_ref[...], dnt,
                         preferred_element_type=f32)

    gSb = gS.astype(bf16)
    P1 = jnp.dot(gSb, wa_ref[...], preferred_element_type=f32)
    P2 = jnp.dot(gD.astype(bf16), wdg_ref[...], preferred_element_type=f32)
    P3 = jnp.dot(edge_ref[...].astype(bf16), weg_ref[...],
                 preferred_element_type=f32)
    m = P1[:, 0:F] + P2 + P3 + bm_ref[...]
    m_out[...] = m.astype(bf16)

    if padded:
        eidx = t * TE + lax.broadcasted_iota(jnp.int32, (TE, 1), 0)
        valid = (eidx < num_edges).astype(f32)
        mv = m * valid
    else:
        valid = None
        mv = m

    st_ref[0:1, :] += jnp.sum(mv, axis=0, keepdims=True)
    st_ref[1:2, :] += jnp.sum(mv * m, axis=0, keepdims=True)

    e = jnp.exp(-jnp.abs(m))
    r = _recip(1.0 + e)
    sigma = jnp.where(m >= 0, r, e * r)
    if padded:
        sigma = sigma * valid

    Bh = P1[:, F:2 * F] + bdu_ref[...]
    contrib = jnp.concatenate([Bh * sigma, sigma], axis=1).astype(bf16)
    acc_ref[...] += jnp.dot(Dt, contrib, preferred_element_type=f32)

    @pl.when(t == last)
    def _finalize():
        xl = jnp.dot(nodebf_ref[...], wsu_ref[...],
                     preferred_element_type=f32) + bsu_ref[...]
        num = acc_ref[:, 0:F]
        den = acc_ref[:, F:2 * F]
        x = xl + num * _recip(den + _DIV_EPS)
        mean = jnp.mean(x, axis=0, keepdims=True)
        var = jnp.mean((x - mean) ** 2, axis=0, keepdims=True)
        xn = (x - mean) * lax.rsqrt(var + _BN_EPS) * gn_ref[...] + bn_ref[...]
        x_out[...] = node_ref[...] + jnp.clip(xn, 0.0, 6.0)

        inv_e = 1.0 / float(num_edges)
        s = st_ref[0:1, :] * inv_e
        q = st_ref[1:2, :] * inv_e
        var_e = q - s * s
        scale = ge_ref[...] * lax.rsqrt(var_e + _BN_EPS)
        ss_out[0:1, :] = scale
        ss_out[1:2, :] = be_ref[...] - s * scale
        ss_out[2:8, :] = jnp.zeros((6, F), f32)


def _edge_out_kernel(edge_ref, m_ref, ss_ref, y_out):
    y_out[...] = edge_ref[...] + jnp.clip(
        m_ref[...].astype(jnp.float32) * ss_ref[0:1, :] + ss_ref[1:2, :],
        0.0, 6.0)


def _pick_tile(n, cap, step=8):
    best = 0
    d = step
    while d <= min(n, cap):
        if n % d == 0:
            best = d
        d += step
    return best


def kernel(node_feats, edge_feats, src, dst,
           w_src_gate, b_src_gate, w_dst_gate, b_dst_gate,
           w_edge_gate, b_edge_gate, w_src_update, b_src_update,
           w_dst_update, b_dst_update,
           bn_nodes_gamma, bn_nodes_beta, bn_edges_gamma, bn_edges_beta):
    f32 = jnp.float32
    bf16 = jnp.bfloat16
    N, F = node_feats.shape
    E = edge_feats.shape[0]

    TE = _pick_tile(E, cap=2048)
    if TE >= 256:
        E_pad, padded = E, False
        edge_p, src_p, dst_p = edge_feats, src, dst
    else:
        TE = 1024
        E_pad = ((E + TE - 1) // TE) * TE
        padded = True
        edge_p = jnp.pad(edge_feats, ((0, E_pad - E), (0, 0)))
        src_p = jnp.pad(src, (0, E_pad - E))
        dst_p = jnp.pad(dst, (0, E_pad - E))
    n_t = E_pad // TE
    src_r = src_p.astype(jnp.int32).reshape(n_t, 1, TE)
    dst_r = dst_p.astype(jnp.int32).reshape(n_t, 1, TE)
    n_tiles = E_pad // TE

    node_bf = node_feats.astype(bf16)
    wa = jnp.concatenate([w_src_gate, w_dst_update], axis=1).astype(bf16)
    bm = b_src_gate + b_dst_gate + b_edge_gate

    m, x, ss = pl.pallas_call(
        functools.partial(_pass1_kernel, num_edges=E, edge_tile=TE,
                          padded=padded),
        out_shape=(jax.ShapeDtypeStruct((E_pad, F), bf16),
                   jax.ShapeDtypeStruct((N, F), f32),
                   jax.ShapeDtypeStruct((8, F), f32)),
        grid=(n_tiles,),
        in_specs=[
            pl.BlockSpec((N, F), lambda t: (0, 0)),
            pl.BlockSpec((N, F), lambda t: (0, 0)),
            pl.BlockSpec((TE, F), lambda t: (t, 0)),
            pl.BlockSpec((1, 1, TE), lambda t: (t, 0, 0)),
            pl.BlockSpec((1, 1, TE), lambda t: (t, 0, 0)),
            pl.BlockSpec((F, 2 * F), lambda t: (0, 0)),
            pl.BlockSpec((F, F), lambda t: (0, 0)),
            pl.BlockSpec((F, F), lambda t: (0, 0)),
            pl.BlockSpec((1, F), lambda t: (0, 0)),
            pl.BlockSpec((1, F), lambda t: (0, 0)),
            pl.BlockSpec((F, F), lambda t: (0, 0)),
            pl.BlockSpec((1, F), lambda t: (0, 0)),
            pl.BlockSpec((1, F), lambda t: (0, 0)),
            pl.BlockSpec((1, F), lambda t: (0, 0)),
            pl.BlockSpec((1, F), lambda t: (0, 0)),
            pl.BlockSpec((1, F), lambda t: (0, 0)),
        ],
        out_specs=(
            pl.BlockSpec((TE, F), lambda t: (t, 0)),
            pl.BlockSpec((N, F), lambda t: (0, 0)),
            pl.BlockSpec((8, F), lambda t: (0, 0)),
        ),
        scratch_shapes=[pltpu.VMEM((N, 2 * F), f32),
                        pltpu.VMEM((8, F), f32)],
        compiler_params=pltpu.CompilerParams(
            dimension_semantics=("arbitrary",),
            vmem_limit_bytes=56 * 1024 * 1024),
    )(node_bf, node_feats, edge_p, src_r, dst_r,
      wa, w_dst_gate.astype(bf16), w_edge_gate.astype(bf16), bm, b_dst_update,
      w_src_update.astype(bf16), b_src_update,
      bn_nodes_gamma, bn_nodes_beta, bn_edges_gamma, bn_edges_beta)

    E4 = E_pad // 4
    edge4 = edge_p.reshape(E4, 4 * F)
    m4 = m.reshape(E4, 4 * F)
    ss4 = jnp.tile(ss, (1, 4))
    tb = _pick_tile(E4, cap=1024)
    if tb == 0:
        tb = E4
    y4 = pl.pallas_call(
        _edge_out_kernel,
        out_shape=jax.ShapeDtypeStruct((E4, 4 * F), f32),
        grid=(E4 // tb,),
        in_specs=[
            pl.BlockSpec((tb, 4 * F), lambda t: (t, 0)),
            pl.BlockSpec((tb, 4 * F), lambda t: (t, 0)),
            pl.BlockSpec((8, 4 * F), lambda t: (0, 0)),
        ],
        out_specs=pl.BlockSpec((tb, 4 * F), lambda t: (t, 0)),
        compiler_params=pltpu.CompilerParams(
            dimension_semantics=("arbitrary",)),
    )(edge4, m4, ss4)
    y = y4.reshape(E_pad, F)[:E]
    return x, y
